# baseline (speedup 1.0000x reference)
import functools
import math

import jax
import jax.numpy as jnp
from jax import lax
from jax.experimental import pallas as pl
from jax.experimental.pallas import tpu as pltpu
from jax.experimental.pallas import tpu_sc as plsc

N_NODES = 10000
D = 128
NB = 10
HID = 100
ACT_C = 1.6765324703310909
C_S = math.sin(math.pi / 8.0)
C_X = math.cos(math.pi / 8.0)

NC = 2
NS = 16
B_EDGE = 128
ROWS_PER_TILE = 624
ROW_TAIL = N_NODES - NS * ROWS_PER_TILE

_HIGH = lax.Precision.HIGHEST
_DN = (((1,), (0,)), ((), ()))


def _node_mm_body(ni_ref, na_ref, wsc_ref, wl1_ref, s_ref, x_ref):
    ni = ni_ref[...]
    na = na_ref[...]
    s_ref[...] = lax.dot_general(ni, wsc_ref[...], _DN, precision=_HIGH) * na
    x_ref[...] = lax.dot_general(ni, wl1_ref[...], _DN, precision=_HIGH) * na


def _radial_body(ele_ref, ea_ref, fc0_ref, fc1_ref, w_ref):
    ele = ele_ref[...].astype(jnp.bfloat16)
    h = lax.dot_general(ele, fc0_ref[...].astype(jnp.bfloat16), _DN,
                        preferred_element_type=jnp.float32)
    h = ACT_C * (h * jax.nn.sigmoid(h))
    w = lax.dot_general(h.astype(jnp.bfloat16),
                        fc1_ref[...].astype(jnp.bfloat16), _DN,
                        preferred_element_type=jnp.float32)
    w_ref[...] = w * ea_ref[...]


def _final_body(s_ref, agg_ref, w2_ref, na_ref, out_ref):
    agg = agg_ref[0] + agg_ref[1]
    x2 = lax.dot_general(agg, w2_ref[...], _DN, precision=_HIGH)
    out_ref[...] = s_ref[...] + x2 * na_ref[...]


def _make_sc_kernel(e: int):
    ept = e // (NC * NS)
    n_full = ept // B_EDGE
    e_tail = ept - n_full * B_EDGE
    assert e_tail % 8 == 0
    mesh = plsc.VectorSubcoreMesh(core_axis_name="c", subcore_axis_name="s")

    @functools.partial(
        pl.kernel,
        out_type=jax.ShapeDtypeStruct((NC, N_NODES, D), jnp.float32),
        mesh=mesh,
        scratch_types=[
            pltpu.VMEM((B_EDGE,), jnp.int32),
            pltpu.VMEM((B_EDGE,), jnp.int32),
            pltpu.VMEM((B_EDGE, D), jnp.float32),
            pltpu.VMEM((B_EDGE, D), jnp.float32),
            pltpu.VMEM((e_tail,), jnp.int32),
            pltpu.VMEM((e_tail,), jnp.int32),
            pltpu.VMEM((e_tail, D), jnp.float32),
            pltpu.VMEM((e_tail, D), jnp.float32),
            pltpu.VMEM_SHARED((N_NODES, D), jnp.float32),
            pltpu.SemaphoreType.DMA,
            pltpu.SemaphoreType.DMA,
            pltpu.SemaphoreType.DMA,
            pltpu.SemaphoreType.DMA,
        ],
    )
    def sc_edges(x_hbm, w_hbm, src_hbm, dst_hbm, agg_hbm,
                 srcv, dstv, wv, gxv, srct, dstt, wt, gxt, aggtab,
                 sem_i, sem_d, sem_g, sem_w):
        c = lax.axis_index("c")
        s = lax.axis_index("s")
        r0 = s * ROWS_PER_TILE
        rows = pl.ds(r0, ROWS_PER_TILE)
        tail = pl.ds(NS * ROWS_PER_TILE, ROW_TAIL)
        zv = jnp.zeros((16,), jnp.float32)
        for r in range(gxt.shape[0]):
            for f in range(D // 16):
                gxt[r, pl.ds(f * 16, 16)] = zv

        zrows = gxt.shape[0]
        @pl.loop(0, ROWS_PER_TILE // zrows)
        def _zero(i):
            pltpu.sync_copy(gxt, aggtab.at[pl.ds(r0 + i * zrows, zrows)])

        @pl.when(s == 0)
        def _tail_in():
            pltpu.sync_copy(gxt, aggtab.at[tail])

        plsc.subcore_barrier()
        base0 = (c * NS + s) * ept

        def _do_chunk(base, blen, isrc, idst, wbuf, gxbuf):
            esl = pl.ds(base, blen)
            pltpu.async_copy(src_hbm.at[esl], isrc, sem_i)
            pltpu.async_copy(dst_hbm.at[esl], idst, sem_d)
            pltpu.async_copy(w_hbm.at[esl], wbuf, sem_w)
            pltpu.make_async_copy(src_hbm.at[esl], isrc, sem_i).wait()
            pltpu.async_copy(x_hbm.at[isrc], gxbuf, sem_g)
            pltpu.make_async_copy(w_hbm.at[esl], wbuf, sem_w).wait()
            pltpu.make_async_copy(x_hbm.at[isrc], gxbuf, sem_g).wait()

            @pl.loop(0, blen, unroll=4)
            def _edge(b):
                for f in range(D // 16):
                    sl = pl.ds(f * 16, 16)
                    wbuf[b, sl] = wbuf[b, sl] * gxbuf[b, sl]

            pltpu.make_async_copy(dst_hbm.at[esl], idst, sem_d).wait()
            pltpu.sync_copy(wbuf, aggtab.at[idst], add=True)

        @pl.loop(0, n_full)
        def _chunk(k):
            _do_chunk(base0 + k * B_EDGE, B_EDGE, srcv, dstv, wv, gxv)

        _do_chunk(base0 + n_full * B_EDGE, e_tail, srct, dstt, wt, gxt)

        plsc.subcore_barrier()
        pltpu.sync_copy(aggtab.at[rows], agg_hbm.at[c].at[rows])

        @pl.when(s == 0)
        def _tail_out():
            pltpu.sync_copy(aggtab.at[tail], agg_hbm.at[c].at[tail])

    return sc_edges


def kernel(node_input, node_attr, edge_src, edge_dst, edge_attr,
           edge_length_embedded, W_sc, W_lin1, W_lin2, fc_w0, fc_w1):
    n, d = node_input.shape
    e = edge_src.shape[0]

    wsc_s = W_sc * (C_S / math.sqrt(D))
    wl1_s = W_lin1 * (1.0 / math.sqrt(D))
    fc0_s = fc_w0 * (1.0 / math.sqrt(NB))
    fc1_s = fc_w1 * (1.0 / (math.sqrt(HID) * math.sqrt(32.0)))
    w2_s = W_lin2 * (C_X / math.sqrt(D))

    nb = 2000
    ngrid = n // nb
    s_scaled, x = pl.pallas_call(
        _node_mm_body,
        grid=(ngrid,),
        in_specs=[
            pl.BlockSpec((nb, d), lambda i: (i, 0)),
            pl.BlockSpec((nb, 1), lambda i: (i, 0)),
            pl.BlockSpec((d, d), lambda i: (0, 0)),
            pl.BlockSpec((d, d), lambda i: (0, 0)),
        ],
        out_specs=[
            pl.BlockSpec((nb, d), lambda i: (i, 0)),
            pl.BlockSpec((nb, d), lambda i: (i, 0)),
        ],
        out_shape=[
            jax.ShapeDtypeStruct((n, d), jnp.float32),
            jax.ShapeDtypeStruct((n, d), jnp.float32),
        ],
    )(node_input, node_attr, wsc_s, wl1_s)

    eb = 12800
    egrid = e // eb
    w_eff = pl.pallas_call(
        _radial_body,
        grid=(egrid,),
        in_specs=[
            pl.BlockSpec((eb, NB), lambda i: (i, 0)),
            pl.BlockSpec((eb, 1), lambda i: (i, 0)),
            pl.BlockSpec((NB, HID), lambda i: (0, 0)),
            pl.BlockSpec((HID, D), lambda i: (0, 0)),
        ],
        out_specs=pl.BlockSpec((eb, D), lambda i: (i, 0)),
        out_shape=jax.ShapeDtypeStruct((e, D), jnp.float32),
    )(edge_length_embedded, edge_attr, fc0_s, fc1_s)

    agg = _make_sc_kernel(e)(x, w_eff, edge_src, edge_dst)

    out = pl.pallas_call(
        _final_body,
        grid=(ngrid,),
        in_specs=[
            pl.BlockSpec((nb, d), lambda i: (i, 0)),
            pl.BlockSpec((NC, nb, d), lambda i: (0, i, 0)),
            pl.BlockSpec((d, d), lambda i: (0, 0)),
            pl.BlockSpec((nb, 1), lambda i: (i, 0)),
        ],
        out_specs=pl.BlockSpec((nb, d), lambda i: (i, 0)),
        out_shape=jax.ShapeDtypeStruct((n, d), jnp.float32),
    )(s_scaled, agg, w2_s, node_attr)
    return out


# speedup vs baseline: 1.4013x; 1.4013x over previous
import functools
import math

import jax
import jax.numpy as jnp
from jax import lax
from jax.experimental import pallas as pl
from jax.experimental.pallas import tpu as pltpu
from jax.experimental.pallas import tpu_sc as plsc

N_NODES = 10000
D = 128
NB = 10
HID = 100
ACT_C = 1.6765324703310909
C_S = math.sin(math.pi / 8.0)
C_X = math.cos(math.pi / 8.0)

NC = 2
NS = 16
B_EDGE = 128
ROWS_PER_TILE = 624
ROW_TAIL = N_NODES - NS * ROWS_PER_TILE

_HIGH = lax.Precision.HIGHEST
_DN = (((1,), (0,)), ((), ()))


def _node_mm_body(ni_ref, na_ref, wsc_ref, wl1_ref, s_ref, x_ref):
    ni = ni_ref[...]
    na = na_ref[...]
    s_ref[...] = lax.dot_general(ni, wsc_ref[...], _DN, precision=_HIGH) * na
    x_ref[...] = lax.dot_general(ni, wl1_ref[...], _DN, precision=_HIGH) * na


def _radial_body(ele_ref, ea_ref, fc0_ref, fc1_ref, w_ref):
    ele = ele_ref[...].astype(jnp.bfloat16)
    h = lax.dot_general(ele, fc0_ref[...].astype(jnp.bfloat16), _DN,
                        preferred_element_type=jnp.float32)
    h = ACT_C * (h * jax.nn.sigmoid(h))
    w = lax.dot_general(h.astype(jnp.bfloat16),
                        fc1_ref[...].astype(jnp.bfloat16), _DN,
                        preferred_element_type=jnp.float32)
    w_ref[...] = w * ea_ref[...]


def _final_body(s_ref, agg_ref, w2_ref, na_ref, out_ref):
    agg = agg_ref[0] + agg_ref[1]
    x2 = lax.dot_general(agg, w2_ref[...], _DN, precision=_HIGH)
    out_ref[...] = s_ref[...] + x2 * na_ref[...]


def _make_sc_kernel(e: int):
    ept = e // (NC * NS)
    n_full = ept // B_EDGE
    e_tail = ept - n_full * B_EDGE
    assert e_tail % 8 == 0
    mesh = plsc.VectorSubcoreMesh(core_axis_name="c", subcore_axis_name="s")

    @functools.partial(
        pl.kernel,
        out_type=jax.ShapeDtypeStruct((NC, N_NODES, D), jnp.float32),
        mesh=mesh,
        scratch_types=[
            pltpu.VMEM((B_EDGE,), jnp.int32),
            pltpu.VMEM((B_EDGE,), jnp.int32),
            pltpu.VMEM((B_EDGE, D), jnp.float32),
            pltpu.VMEM((B_EDGE, D), jnp.float32),
            pltpu.VMEM((e_tail,), jnp.int32),
            pltpu.VMEM((e_tail,), jnp.int32),
            pltpu.VMEM((e_tail, D), jnp.float32),
            pltpu.VMEM((e_tail, D), jnp.float32),
            pltpu.VMEM_SHARED((N_NODES, D), jnp.float32),
            pltpu.SemaphoreType.DMA,
            pltpu.SemaphoreType.DMA,
            pltpu.SemaphoreType.DMA,
            pltpu.SemaphoreType.DMA,
        ],
    )
    def sc_edges(x_hbm, w_hbm, src_hbm, dst_hbm, agg_hbm,
                 srcv, dstv, wv, gxv, srct, dstt, wt, gxt, aggtab,
                 sem_i, sem_d, sem_g, sem_w):
        c = lax.axis_index("c")
        s = lax.axis_index("s")
        r0 = s * ROWS_PER_TILE
        rows = pl.ds(r0, ROWS_PER_TILE)
        tail = pl.ds(NS * ROWS_PER_TILE, ROW_TAIL)
        zv = jnp.zeros((16,), jnp.float32)
        for r in range(gxt.shape[0]):
            for f in range(D // 16):
                gxt[r, pl.ds(f * 16, 16)] = zv

        zrows = gxt.shape[0]
        @pl.loop(0, ROWS_PER_TILE // zrows)
        def _zero(i):
            pltpu.sync_copy(gxt, aggtab.at[pl.ds(r0 + i * zrows, zrows)])

        @pl.when(s == 0)
        def _tail_in():
            pltpu.sync_copy(gxt, aggtab.at[tail])

        plsc.subcore_barrier()
        base0 = (c * NS + s) * ept

        def _do_chunk(base, blen, isrc, idst, wbuf, gxbuf):
            esl = pl.ds(base, blen)
            pltpu.async_copy(src_hbm.at[esl], isrc, sem_i)
            pltpu.async_copy(dst_hbm.at[esl], idst, sem_d)
            pltpu.async_copy(w_hbm.at[esl], wbuf, sem_w)
            pltpu.make_async_copy(src_hbm.at[esl], isrc, sem_i).wait()
            pltpu.async_copy(x_hbm.at[isrc], gxbuf, sem_g)
            pltpu.make_async_copy(w_hbm.at[esl], wbuf, sem_w).wait()
            pltpu.make_async_copy(x_hbm.at[isrc], gxbuf, sem_g).wait()

            @pl.loop(0, blen)
            def _edge(b):
                for f in range(D // 16):
                    sl = pl.ds(f * 16, 16)
                    wbuf[b, sl] = wbuf[b, sl] * gxbuf[b, sl]

            pltpu.make_async_copy(dst_hbm.at[esl], idst, sem_d).wait()
            pltpu.sync_copy(wbuf, aggtab.at[idst], add=True)

        @pl.loop(0, n_full)
        def _chunk(k):
            _do_chunk(base0 + k * B_EDGE, B_EDGE, srcv, dstv, wv, gxv)

        _do_chunk(base0 + n_full * B_EDGE, e_tail, srct, dstt, wt, gxt)

        plsc.subcore_barrier()
        pltpu.sync_copy(aggtab.at[rows], agg_hbm.at[c].at[rows])

        @pl.when(s == 0)
        def _tail_out():
            pltpu.sync_copy(aggtab.at[tail], agg_hbm.at[c].at[tail])

    return sc_edges


def kernel(node_input, node_attr, edge_src, edge_dst, edge_attr,
           edge_length_embedded, W_sc, W_lin1, W_lin2, fc_w0, fc_w1):
    n, d = node_input.shape
    e = edge_src.shape[0]

    wsc_s = W_sc * (C_S / math.sqrt(D))
    wl1_s = W_lin1 * (1.0 / math.sqrt(D))
    fc0_s = fc_w0 * (1.0 / math.sqrt(NB))
    fc1_s = fc_w1 * (1.0 / (math.sqrt(HID) * math.sqrt(32.0)))
    w2_s = W_lin2 * (C_X / math.sqrt(D))

    nb = 2000
    ngrid = n // nb
    s_scaled, x = pl.pallas_call(
        _node_mm_body,
        grid=(ngrid,),
        in_specs=[
            pl.BlockSpec((nb, d), lambda i: (i, 0)),
            pl.BlockSpec((nb, 1), lambda i: (i, 0)),
            pl.BlockSpec((d, d), lambda i: (0, 0)),
            pl.BlockSpec((d, d), lambda i: (0, 0)),
        ],
        out_specs=[
            pl.BlockSpec((nb, d), lambda i: (i, 0)),
            pl.BlockSpec((nb, d), lambda i: (i, 0)),
        ],
        out_shape=[
            jax.ShapeDtypeStruct((n, d), jnp.float32),
            jax.ShapeDtypeStruct((n, d), jnp.float32),
        ],
    )(node_input, node_attr, wsc_s, wl1_s)

    eb = 6400
    egrid = e // eb
    w_eff = pl.pallas_call(
        _radial_body,
        grid=(egrid,),
        in_specs=[
            pl.BlockSpec((eb, NB), lambda i: (i, 0)),
            pl.BlockSpec((eb, 1), lambda i: (i, 0)),
            pl.BlockSpec((NB, HID), lambda i: (0, 0)),
            pl.BlockSpec((HID, D), lambda i: (0, 0)),
        ],
        out_specs=pl.BlockSpec((eb, D), lambda i: (i, 0)),
        out_shape=jax.ShapeDtypeStruct((e, D), jnp.float32),
    )(edge_length_embedded, edge_attr, fc0_s, fc1_s)

    agg = _make_sc_kernel(e)(x, w_eff, edge_src, edge_dst)

    out = pl.pallas_call(
        _final_body,
        grid=(ngrid,),
        in_specs=[
            pl.BlockSpec((nb, d), lambda i: (i, 0)),
            pl.BlockSpec((NC, nb, d), lambda i: (0, i, 0)),
            pl.BlockSpec((d, d), lambda i: (0, 0)),
            pl.BlockSpec((nb, 1), lambda i: (i, 0)),
        ],
        out_specs=pl.BlockSpec((nb, d), lambda i: (i, 0)),
        out_shape=jax.ShapeDtypeStruct((n, d), jnp.float32),
    )(s_scaled, agg, w2_s, node_attr)
    return out
